# baseline (speedup 1.0000x reference)
import functools

import jax
import jax.numpy as jnp
from jax import lax
from jax.experimental import pallas as pl
from jax.experimental.pallas import tpu as pltpu
from jax.experimental.pallas import tpu_sc as plsc

U_RES = 400
CH = 8
H_RES = 50
ANG = 8
HID = 32
B = 524288

NC = 2
NS = 16
NW = NC * NS
QPW = B // NW
L = 16
CHUNK_Q = 1024
GPC = CHUNK_Q // L
NCHUNK = QPW // CHUNK_Q
NGRP = QPW // L

HD_WORDS = 2 * H_RES * (H_RES + 1) * ANG
D_OFF = H_RES * (H_RES + 1) * ANG

LA = 4
NSLOT = 8

MLP_BN = 8192


BSEG = 1000


def _build_body(u_hbm, up_hbm, rowin, rowout, sem):
    wid = lax.axis_index("s") * NC + lax.axis_index("c")
    rpw = (U_RES * U_RES) // NW
    r0 = wid * rpw
    lane = lax.iota(jnp.int32, L)

    for s in range(rpw // BSEG):
        seg0 = r0 + s * BSEG
        pltpu.async_copy(u_hbm.at[pl.ds(seg0 * CH, (BSEG + 1) * CH)],
                         rowin, sem).wait()

        def px(j, carry):
            r = seg0 + j
            xcoord = lax.rem(r, U_RES)
            msk = jax.lax.select(xcoord == U_RES - 1,
                                 jnp.int32(CH - 1), jnp.int32(2 * CH - 1))
            idx = j * CH + (lane & msk)
            v = plsc.load_gather(rowin, [idx])
            plsc.store_scatter(rowout, [j * (2 * CH) + lane], v)
            return carry

        lax.fori_loop(0, BSEG, px, 0)
        pltpu.sync_copy(rowout,
                        up_hbm.at[pl.ds(seg0 * (2 * CH), BSEG * 2 * CH)])


def _build_upairs(u_flat):
    mesh = plsc.VectorSubcoreMesh(core_axis_name="c", subcore_axis_name="s")
    f = pl.kernel(
        _build_body,
        out_type=jax.ShapeDtypeStruct((U_RES * U_RES * 2 * CH,), jnp.float32),
        mesh=mesh,
        compiler_params=pltpu.CompilerParams(
            needs_layout_passes=False, use_tc_tiling_on_sc=False
        ),
        scratch_types=[
            pltpu.VMEM(((BSEG + 1) * CH,), jnp.float32),
            pltpu.VMEM((BSEG * 2 * CH,), jnp.float32),
            pltpu.SemaphoreType.DMA,
        ],
    )
    return f(u_flat)


def _sc_body(x_hbm, uplane_hbm, hd_hbm, feat_hbm,
             hdbuf, xbuf, featbuf, gbuf, idxring, fracring,
             semx, semg, semf):
    wid = lax.axis_index("s") * NC + lax.axis_index("c")
    qbase = wid * QPW

    pltpu.sync_copy(hd_hbm, hdbuf)

    lane = lax.iota(jnp.int32, L)

    pltpu.async_copy(
        x_hbm.at[pl.ds(qbase * 6, CHUNK_Q * 6)],
        xbuf.at[pl.ds(0, CHUNK_Q * 6)], semx)

    def xcomp(par, gl, comp):
        idx = (gl * L + lane) * 6 + (par * (CHUNK_Q * 6) + comp)
        return plsc.load_gather(xbuf, [idx])

    def bil_weights(urf, vrf):
        w10 = urf * (1.0 - vrf)
        w01 = (1.0 - urf) * vrf
        w11 = urf * vrf
        w00 = (1.0 - urf) * (1.0 - vrf)
        return w00, w10, w01, w11

    ROWSTRIDE = (H_RES + 1) * ANG

    def cconst(c):
        return jnp.full((L,), c, jnp.int32)

    def grp(g, carry):
        kc = g >> 6
        gl = g & 63
        par = kc & 1

        @pl.when(g < NGRP)
        def _():
            @pl.when(gl == 0)
            def _():
                pltpu.make_async_copy(
                    x_hbm.at[pl.ds((qbase + kc * CHUNK_Q) * 6, CHUNK_Q * 6)],
                    xbuf.at[pl.ds(par * (CHUNK_Q * 6), CHUNK_Q * 6)],
                    semx).wait()

            u0 = xcomp(par, gl, 0)
            v0 = xcomp(par, gl, 1)
            uu = u0 * jnp.float32(U_RES - 1)
            vv = v0 * jnp.float32(U_RES - 1)
            xi = jnp.minimum(uu.astype(jnp.int32), U_RES - 1)
            yi = jnp.minimum(vv.astype(jnp.int32), U_RES - 1)
            ur = uu - xi.astype(jnp.float32)
            vr = vv - yi.astype(jnp.float32)
            slot = g & (NSLOT - 1)
            sbase = slot * (2 * L)
            row0 = yi * U_RES + xi
            row1 = jnp.minimum(yi + 1, U_RES - 1) * U_RES + xi
            for j, rows in enumerate((row0, row1)):
                plsc.store_scatter(idxring, [lane + (sbase + j * L)], rows)
                pltpu.async_copy(uplane_hbm.at[rows],
                                 gbuf.at[2 * slot + j],
                                 semg.at[2 * slot + j])
            plsc.store_scatter(fracring, [lane + slot * (2 * L)], ur)
            plsc.store_scatter(fracring, [lane + (slot * (2 * L) + L)], vr)

            @pl.when(jnp.logical_and(gl == LA, kc + 1 < NCHUNK))
            def _():
                pltpu.async_copy(
                    x_hbm.at[pl.ds((qbase + (kc + 1) * CHUNK_Q) * 6,
                                   CHUNK_Q * 6)],
                    xbuf.at[pl.ds((1 - par) * (CHUNK_Q * 6), CHUNK_Q * 6)],
                    semx)

        @pl.when(g >= LA)
        def _():
            cg = g - LA
            ckc = cg >> 6
            cgl = cg & 63
            cpar = ckc & 1
            slot = cg & (NSLOT - 1)
            sbase = slot * (2 * L)

            @pl.when(jnp.logical_and(cgl == 0, ckc >= 2))
            def _():
                for r in range(12):
                    pltpu.make_async_copy(
                        featbuf.at[cpar, r],
                        feat_hbm.at[r, pl.ds(qbase + (ckc - 2) * CHUNK_Q,
                                             CHUNK_Q)],
                        semf.at[cpar]).wait()

            qvec = cgl * L + lane
            fb = featbuf.at[cpar]

            hphi = xcomp(cpar, cgl, 3)
            hth = xcomp(cpar, cgl, 2)
            hu = hphi * jnp.float32(H_RES)
            hv = hth * jnp.float32(H_RES - 1)
            hxi = jnp.minimum(hu.astype(jnp.int32), H_RES - 1)
            hyi = jnp.minimum(hv.astype(jnp.int32), H_RES - 2)
            hur = hu - hxi.astype(jnp.float32)
            hvr = hv - hyi.astype(jnp.float32)
            hbase = (hyi * (H_RES + 1) + hxi) * ANG

            dphi = xcomp(cpar, cgl, 5)
            dth = xcomp(cpar, cgl, 4)
            du = dphi * jnp.float32(H_RES)
            dv = dth * jnp.float32(H_RES - 1)
            dxi = jnp.minimum(du.astype(jnp.int32), H_RES - 1)
            dyi = jnp.minimum(dv.astype(jnp.int32), H_RES - 2)
            dur = du - dxi.astype(jnp.float32)
            dvr = dv - dyi.astype(jnp.float32)
            dbase = (dyi * (H_RES + 1) + dxi) * ANG + D_OFF

            hw = bil_weights(hur, hvr)
            dw = bil_weights(dur, dvr)

            def hd_chan(base, w, c):
                p00 = plsc.load_gather(hdbuf, [base + c])
                p10 = plsc.load_gather(hdbuf, [base + (ANG + c)])
                p01 = plsc.load_gather(hdbuf, [base + (ROWSTRIDE + c)])
                p11 = plsc.load_gather(hdbuf, [base + (ROWSTRIDE + ANG + c)])
                return w[0] * p00 + w[1] * p10 + w[2] * p01 + w[3] * p11

            def pack_store(cpair, fe, fo):
                w = plsc.bitcast(
                    plsc.pack(fe, fo, format=plsc.PackFormat.INTERLEAVED),
                    jnp.int32)
                plsc.store_scatter(fb, [cconst(cpair), qvec], w)

            crows = [plsc.load_gather(idxring, [lane + (sbase + j * L)])
                     for j in range(2)]
            cur = plsc.load_gather(fracring, [lane + slot * (2 * L)])
            cvr = plsc.load_gather(fracring, [lane + (slot * (2 * L) + L)])
            for j in range(2):
                pltpu.make_async_copy(uplane_hbm.at[crows[j]],
                                      gbuf.at[2 * slot + j],
                                      semg.at[2 * slot + j]).wait()
            uw = bil_weights(cur, cvr)

            def u_chan(c):
                cc = cconst(c)
                cc8 = cconst(CH + c)
                p00 = plsc.load_gather(gbuf.at[2 * slot + 0], [lane, cc])
                p10 = plsc.load_gather(gbuf.at[2 * slot + 0], [lane, cc8])
                p01 = plsc.load_gather(gbuf.at[2 * slot + 1], [lane, cc])
                p11 = plsc.load_gather(gbuf.at[2 * slot + 1], [lane, cc8])
                return uw[0] * p00 + uw[1] * p10 + uw[2] * p01 + uw[3] * p11

            for cp in range(4):
                pack_store(cp, u_chan(2 * cp), u_chan(2 * cp + 1))
            for cp in range(4):
                pack_store(4 + cp, hd_chan(hbase, hw, 2 * cp),
                           hd_chan(hbase, hw, 2 * cp + 1))
            for cp in range(4):
                pack_store(8 + cp, hd_chan(dbase, dw, 2 * cp),
                           hd_chan(dbase, dw, 2 * cp + 1))

            @pl.when(cgl == GPC - 1)
            def _():
                for r in range(12):
                    pltpu.async_copy(
                        featbuf.at[cpar, r],
                        feat_hbm.at[r, pl.ds(qbase + ckc * CHUNK_Q, CHUNK_Q)],
                        semf.at[cpar])

        return carry

    lax.fori_loop(0, NGRP + LA, grp, 0)

    for ckc in (NCHUNK - 2, NCHUNK - 1):
        cpar = ckc & 1
        for r in range(12):
            pltpu.make_async_copy(
                featbuf.at[cpar, r],
                feat_hbm.at[r, pl.ds(qbase + ckc * CHUNK_Q, CHUNK_Q)],
                semf.at[cpar]).wait()


def _sc_features(x_flat, uplane8, hd_flat):
    mesh = plsc.VectorSubcoreMesh(core_axis_name="c", subcore_axis_name="s")
    f = pl.kernel(
        _sc_body,
        out_type=jax.ShapeDtypeStruct((12, B), jnp.int32),
        mesh=mesh,
        compiler_params=pltpu.CompilerParams(
            needs_layout_passes=False, use_tc_tiling_on_sc=False
        ),
        scratch_types=[
            pltpu.VMEM((HD_WORDS,), jnp.float32),
            pltpu.VMEM((2 * CHUNK_Q * 6,), jnp.float32),
            pltpu.VMEM((2, 12, CHUNK_Q), jnp.int32),
            pltpu.VMEM((2 * NSLOT, L, 2 * CH), jnp.float32),
            pltpu.VMEM((NSLOT * 2 * L,), jnp.int32),
            pltpu.VMEM((NSLOT * 2 * L,), jnp.float32),
            pltpu.SemaphoreType.DMA,
            pltpu.SemaphoreType.DMA((2 * NSLOT,)),
            pltpu.SemaphoreType.DMA((2,)),
        ],
    )
    return f(x_flat, uplane8, hd_flat)


def _mlp_body(ft_ref, w0_ref, w1_ref, w2_ref, w3_ref, out_ref):
    fb = pltpu.bitcast(ft_ref[...], jnp.bfloat16)
    y = jnp.maximum(
        jnp.dot(w0_ref[...], fb, preferred_element_type=jnp.float32), 0.0)
    y = jnp.maximum(
        jnp.dot(w1_ref[...], y.astype(jnp.bfloat16),
                preferred_element_type=jnp.float32), 0.0)
    y = jnp.maximum(
        jnp.dot(w2_ref[...], y.astype(jnp.bfloat16),
                preferred_element_type=jnp.float32), 0.0)
    out_ref[...] = jnp.dot(w3_ref[...], y.astype(jnp.bfloat16),
                           preferred_element_type=jnp.float32)


def _mlp_t(ft, w0, w1, w2, w3p):
    grid = B // MLP_BN
    return pl.pallas_call(
        _mlp_body,
        grid=(grid,),
        in_specs=[
            pl.BlockSpec((12, MLP_BN), lambda i: (0, i)),
            pl.BlockSpec((HID, 24), lambda i: (0, 0)),
            pl.BlockSpec((HID, HID), lambda i: (0, 0)),
            pl.BlockSpec((HID, HID), lambda i: (0, 0)),
            pl.BlockSpec((8, HID), lambda i: (0, 0)),
        ],
        out_specs=pl.BlockSpec((8, MLP_BN), lambda i: (0, i)),
        out_shape=jax.ShapeDtypeStruct((8, B), jnp.float32),
    )(ft, w0, w1, w2, w3p)


def kernel(x, u_plane, h_plane, d_plane, W0, W1, W2, W3):
    h_pad = jnp.concatenate([h_plane, h_plane[:, :1, :]], axis=1).reshape(-1)
    d_pad = jnp.concatenate([d_plane, d_plane[:, :1, :]], axis=1).reshape(-1)
    hd_flat = jnp.concatenate([h_pad, d_pad])

    u_flat = u_plane.reshape(-1)
    u_flat = jnp.concatenate([u_flat, u_flat[-CH:]])
    upairs = _build_upairs(u_flat).reshape(U_RES * U_RES, 2 * CH)

    ft = _sc_features(x.reshape(-1), upairs, hd_flat)

    w0 = W0.astype(jnp.bfloat16)
    w1 = W1.astype(jnp.bfloat16)
    w2 = W2.astype(jnp.bfloat16)
    w3p = jnp.pad(W3, ((0, 5), (0, 0))).astype(jnp.bfloat16)
    out_t = _mlp_t(ft, w0, w1, w2, w3p)
    return out_t[:3].T


# speedup vs baseline: 1.0478x; 1.0478x over previous
import functools

import jax
import jax.numpy as jnp
from jax import lax
from jax.experimental import pallas as pl
from jax.experimental.pallas import tpu as pltpu
from jax.experimental.pallas import tpu_sc as plsc

U_RES = 400
CH = 8
H_RES = 50
ANG = 8
HID = 32
B = 524288

NC = 2
NS = 16
NW = NC * NS
QPW = B // NW
L = 16
CHUNK_Q = 1024
GPC = CHUNK_Q // L
NCHUNK = QPW // CHUNK_Q
NGRP = QPW // L

HD_WORDS = 2 * H_RES * (H_RES + 1) * ANG
D_OFF = H_RES * (H_RES + 1) * ANG

LA = 4
NSLOT = 8

MLP_BN = 8192


def _sc_body(x_hbm, uplane_hbm, hd_hbm, feat_hbm,
             hdbuf, xbuf, featbuf, gbuf, idxring, fracring,
             semx, semg, semf):
    wid = lax.axis_index("s") * NC + lax.axis_index("c")
    qbase = wid * QPW

    pltpu.sync_copy(hd_hbm, hdbuf)

    lane = lax.iota(jnp.int32, L)

    pltpu.async_copy(
        x_hbm.at[pl.ds(qbase * 6, CHUNK_Q * 6)],
        xbuf.at[pl.ds(0, CHUNK_Q * 6)], semx)

    def xcomp(par, gl, comp):
        idx = (gl * L + lane) * 6 + (par * (CHUNK_Q * 6) + comp)
        return plsc.load_gather(xbuf, [idx])

    def bil_weights(urf, vrf):
        w10 = urf * (1.0 - vrf)
        w01 = (1.0 - urf) * vrf
        w11 = urf * vrf
        w00 = (1.0 - urf) * (1.0 - vrf)
        return w00, w10, w01, w11

    ROWSTRIDE = (H_RES + 1) * ANG

    def cconst(c):
        return jnp.full((L,), c, jnp.int32)

    def one_group(g):
        kc = g >> 6
        gl = g & 63
        par = kc & 1

        @pl.when(g < NGRP)
        def _():
            @pl.when(gl == 0)
            def _():
                pltpu.make_async_copy(
                    x_hbm.at[pl.ds((qbase + kc * CHUNK_Q) * 6, CHUNK_Q * 6)],
                    xbuf.at[pl.ds(par * (CHUNK_Q * 6), CHUNK_Q * 6)],
                    semx).wait()

            u0 = xcomp(par, gl, 0)
            v0 = xcomp(par, gl, 1)
            uu = u0 * jnp.float32(U_RES - 1)
            vv = v0 * jnp.float32(U_RES - 1)
            xi = jnp.minimum(uu.astype(jnp.int32), U_RES - 1)
            yi = jnp.minimum(vv.astype(jnp.int32), U_RES - 1)
            ur = uu - xi.astype(jnp.float32)
            vr = vv - yi.astype(jnp.float32)
            slot = g & (NSLOT - 1)
            sbase = slot * (2 * L)
            row0 = yi * U_RES + xi
            row1 = jnp.minimum(yi + 1, U_RES - 1) * U_RES + xi
            for j, rows in enumerate((row0, row1)):
                plsc.store_scatter(idxring, [lane + (sbase + j * L)], rows)
                pltpu.async_copy(uplane_hbm.at[rows],
                                 gbuf.at[2 * slot + j],
                                 semg.at[2 * slot + j])
            plsc.store_scatter(fracring, [lane + slot * (2 * L)], ur)
            plsc.store_scatter(fracring, [lane + (slot * (2 * L) + L)], vr)

            @pl.when(jnp.logical_and(gl == LA, kc + 1 < NCHUNK))
            def _():
                pltpu.async_copy(
                    x_hbm.at[pl.ds((qbase + (kc + 1) * CHUNK_Q) * 6,
                                   CHUNK_Q * 6)],
                    xbuf.at[pl.ds((1 - par) * (CHUNK_Q * 6), CHUNK_Q * 6)],
                    semx)

        @pl.when(g >= LA)
        def _():
            cg = g - LA
            ckc = cg >> 6
            cgl = cg & 63
            cpar = ckc & 1
            slot = cg & (NSLOT - 1)
            sbase = slot * (2 * L)

            @pl.when(jnp.logical_and(cgl == 0, ckc >= 2))
            def _():
                for r in range(12):
                    pltpu.make_async_copy(
                        featbuf.at[cpar, r],
                        feat_hbm.at[r, pl.ds(qbase + (ckc - 2) * CHUNK_Q,
                                             CHUNK_Q)],
                        semf.at[cpar]).wait()

            qvec = cgl * L + lane
            fb = featbuf.at[cpar]

            hphi = xcomp(cpar, cgl, 3)
            hth = xcomp(cpar, cgl, 2)
            hu = hphi * jnp.float32(H_RES)
            hv = hth * jnp.float32(H_RES - 1)
            hxi = jnp.minimum(hu.astype(jnp.int32), H_RES - 1)
            hyi = jnp.minimum(hv.astype(jnp.int32), H_RES - 2)
            hur = hu - hxi.astype(jnp.float32)
            hvr = hv - hyi.astype(jnp.float32)
            hbase = (hyi * (H_RES + 1) + hxi) * ANG

            dphi = xcomp(cpar, cgl, 5)
            dth = xcomp(cpar, cgl, 4)
            du = dphi * jnp.float32(H_RES)
            dv = dth * jnp.float32(H_RES - 1)
            dxi = jnp.minimum(du.astype(jnp.int32), H_RES - 1)
            dyi = jnp.minimum(dv.astype(jnp.int32), H_RES - 2)
            dur = du - dxi.astype(jnp.float32)
            dvr = dv - dyi.astype(jnp.float32)
            dbase = (dyi * (H_RES + 1) + dxi) * ANG + D_OFF

            hw = bil_weights(hur, hvr)
            dw = bil_weights(dur, dvr)

            def hd_chan(base, w, c):
                p00 = plsc.load_gather(hdbuf, [base + c])
                p10 = plsc.load_gather(hdbuf, [base + (ANG + c)])
                p01 = plsc.load_gather(hdbuf, [base + (ROWSTRIDE + c)])
                p11 = plsc.load_gather(hdbuf, [base + (ROWSTRIDE + ANG + c)])
                return w[0] * p00 + w[1] * p10 + w[2] * p01 + w[3] * p11

            def pack_store(cpair, fe, fo):
                w = plsc.bitcast(
                    plsc.pack(fe, fo, format=plsc.PackFormat.INTERLEAVED),
                    jnp.int32)
                plsc.store_scatter(fb, [cconst(cpair), qvec], w)

            crows = [plsc.load_gather(idxring, [lane + (sbase + j * L)])
                     for j in range(2)]
            cur = plsc.load_gather(fracring, [lane + slot * (2 * L)])
            cvr = plsc.load_gather(fracring, [lane + (slot * (2 * L) + L)])
            for j in range(2):
                pltpu.make_async_copy(uplane_hbm.at[crows[j]],
                                      gbuf.at[2 * slot + j],
                                      semg.at[2 * slot + j]).wait()
            uw = bil_weights(cur, cvr)

            def u_chan(c):
                cc = cconst(c)
                cc8 = cconst(CH + c)
                p00 = plsc.load_gather(gbuf.at[2 * slot + 0], [lane, cc])
                p10 = plsc.load_gather(gbuf.at[2 * slot + 0], [lane, cc8])
                p01 = plsc.load_gather(gbuf.at[2 * slot + 1], [lane, cc])
                p11 = plsc.load_gather(gbuf.at[2 * slot + 1], [lane, cc8])
                return uw[0] * p00 + uw[1] * p10 + uw[2] * p01 + uw[3] * p11

            for cp in range(4):
                pack_store(cp, u_chan(2 * cp), u_chan(2 * cp + 1))
            for cp in range(4):
                pack_store(4 + cp, hd_chan(hbase, hw, 2 * cp),
                           hd_chan(hbase, hw, 2 * cp + 1))
            for cp in range(4):
                pack_store(8 + cp, hd_chan(dbase, dw, 2 * cp),
                           hd_chan(dbase, dw, 2 * cp + 1))

            @pl.when(cgl == GPC - 1)
            def _():
                for r in range(12):
                    pltpu.async_copy(
                        featbuf.at[cpar, r],
                        feat_hbm.at[r, pl.ds(qbase + ckc * CHUNK_Q, CHUNK_Q)],
                        semf.at[cpar])

    def grp(gi, carry):
        one_group(gi * 2)
        one_group(gi * 2 + 1)
        return carry

    lax.fori_loop(0, (NGRP + LA) // 2, grp, 0)

    for ckc in (NCHUNK - 2, NCHUNK - 1):
        cpar = ckc & 1
        for r in range(12):
            pltpu.make_async_copy(
                featbuf.at[cpar, r],
                feat_hbm.at[r, pl.ds(qbase + ckc * CHUNK_Q, CHUNK_Q)],
                semf.at[cpar]).wait()


def _sc_features(x_flat, uplane8, hd_flat):
    mesh = plsc.VectorSubcoreMesh(core_axis_name="c", subcore_axis_name="s")
    f = pl.kernel(
        _sc_body,
        out_type=jax.ShapeDtypeStruct((12, B), jnp.int32),
        mesh=mesh,
        compiler_params=pltpu.CompilerParams(
            needs_layout_passes=False, use_tc_tiling_on_sc=False
        ),
        scratch_types=[
            pltpu.VMEM((HD_WORDS,), jnp.float32),
            pltpu.VMEM((2 * CHUNK_Q * 6,), jnp.float32),
            pltpu.VMEM((2, 12, CHUNK_Q), jnp.int32),
            pltpu.VMEM((2 * NSLOT, L, 2 * CH), jnp.float32),
            pltpu.VMEM((NSLOT * 2 * L,), jnp.int32),
            pltpu.VMEM((NSLOT * 2 * L,), jnp.float32),
            pltpu.SemaphoreType.DMA,
            pltpu.SemaphoreType.DMA((2 * NSLOT,)),
            pltpu.SemaphoreType.DMA((2,)),
        ],
    )
    return f(x_flat, uplane8, hd_flat)


def _mlp_body(ft_ref, w0_ref, w1_ref, w2_ref, w3_ref, out_ref):
    fb = pltpu.bitcast(ft_ref[...], jnp.bfloat16)
    y = jnp.maximum(
        jnp.dot(w0_ref[...], fb, preferred_element_type=jnp.float32), 0.0)
    y = jnp.maximum(
        jnp.dot(w1_ref[...], y.astype(jnp.bfloat16),
                preferred_element_type=jnp.float32), 0.0)
    y = jnp.maximum(
        jnp.dot(w2_ref[...], y.astype(jnp.bfloat16),
                preferred_element_type=jnp.float32), 0.0)
    out_ref[...] = jnp.dot(w3_ref[...], y.astype(jnp.bfloat16),
                           preferred_element_type=jnp.float32)


def _mlp_t(ft, w0, w1, w2, w3p):
    grid = B // MLP_BN
    return pl.pallas_call(
        _mlp_body,
        grid=(grid,),
        in_specs=[
            pl.BlockSpec((12, MLP_BN), lambda i: (0, i)),
            pl.BlockSpec((HID, 24), lambda i: (0, 0)),
            pl.BlockSpec((HID, HID), lambda i: (0, 0)),
            pl.BlockSpec((HID, HID), lambda i: (0, 0)),
            pl.BlockSpec((8, HID), lambda i: (0, 0)),
        ],
        out_specs=pl.BlockSpec((8, MLP_BN), lambda i: (0, i)),
        out_shape=jax.ShapeDtypeStruct((8, B), jnp.float32),
    )(ft, w0, w1, w2, w3p)


def kernel(x, u_plane, h_plane, d_plane, W0, W1, W2, W3):
    h_pad = jnp.concatenate([h_plane, h_plane[:, :1, :]], axis=1).reshape(-1)
    d_pad = jnp.concatenate([d_plane, d_plane[:, :1, :]], axis=1).reshape(-1)
    hd_flat = jnp.concatenate([h_pad, d_pad])

    u_next = jnp.concatenate([u_plane[:, 1:, :], u_plane[:, -1:, :]], axis=1)
    upairs = jnp.concatenate([u_plane, u_next], axis=2).reshape(
        U_RES * U_RES, 2 * CH)

    ft = _sc_features(x.reshape(-1), upairs, hd_flat)

    w0 = W0.astype(jnp.bfloat16)
    w1 = W1.astype(jnp.bfloat16)
    w2 = W2.astype(jnp.bfloat16)
    w3p = jnp.pad(W3, ((0, 5), (0, 0))).astype(jnp.bfloat16)
    out_t = _mlp_t(ft, w0, w1, w2, w3p)
    return out_t[:3].T
